# baseline (speedup 1.0000x reference)
import functools

import jax
import jax.numpy as jnp
from jax import lax
from jax.experimental import pallas as pl
from jax.experimental.pallas import tpu as pltpu
from jax.experimental.pallas import tpu_sc as plsc

_NC = 2
_NS = 16
_BLK = 2048


def _sc_gather(table, idx, n_rows, dim):
    nw = _NC * _NS
    b_per_w = n_rows // nw
    n_ch = 4
    cs = b_per_w // n_ch
    mesh = plsc.VectorSubcoreMesh(core_axis_name="c", subcore_axis_name="s")

    @functools.partial(
        pl.kernel,
        mesh=mesh,
        out_type=jax.ShapeDtypeStruct((n_rows, dim), jnp.float32),
        scratch_types=(
            [pltpu.VMEM((n_ch, cs), jnp.int32)]
            + [pltpu.VMEM((cs, dim), jnp.float32) for _ in range(n_ch)]
            + [pltpu.SemaphoreType.DMA for _ in range(2 * n_ch)]
        ),
    )
    def gather_kernel(table_hbm, idx_hbm, out_hbm, idx_v, *bufs_sems):
        bufs = bufs_sems[:n_ch]
        semg = bufs_sems[n_ch:2 * n_ch]
        semo = bufs_sems[2 * n_ch:]
        wid = lax.axis_index("s") * _NC + lax.axis_index("c")
        base = wid * b_per_w
        gath = []
        for c in range(n_ch):
            pltpu.sync_copy(idx_hbm.at[pl.ds(base + c * cs, cs)], idx_v.at[c])
            gath.append(
                pltpu.async_copy(table_hbm.at[idx_v.at[c]], bufs[c], semg[c]))
        outs = []
        for c in range(n_ch):
            gath[c].wait()
            outs.append(
                pltpu.async_copy(bufs[c], out_hbm.at[pl.ds(base + c * cs, cs)],
                                 semo[c]))
        for c in range(n_ch):
            outs[c].wait()

    return gather_kernel(table, idx)


def _main_kernel(gr_ref, a_ref, xm_ref, ct_ref, w1_ref, w1m_ref,
                 b1_ref, w2_ref, b2_ref, out_ref):
    pcomb = lax.dot_general(
        ct_ref[...], w1_ref[...], (((1,), (1,)), ((), ())),
        preferred_element_type=jnp.float32,
    )
    row = lax.broadcasted_iota(jnp.int32, pcomb.shape, 0)
    pcomb = pcomb + jnp.where(row < 16, b1_ref[...], 0.0)

    gr = gr_ref[...]
    a = a_ref[...]
    lane = lax.broadcasted_iota(jnp.int32, (_BLK, 128), 1)
    onehot = (lane == gr[:, None]) | (lane == a[:, None] + 16)
    m = onehot.astype(jnp.float32)
    h = lax.dot_general(
        m, pcomb, (((1,), (0,)), ((), ())),
        preferred_element_type=jnp.float32,
    )
    h = h + lax.dot_general(
        xm_ref[...], w1m_ref[...], (((1,), (1,)), ((), ())),
        preferred_element_type=jnp.float32,
    )
    h = jnp.maximum(h, 0.0)
    o = lax.dot_general(
        w2_ref[...], h, (((1,), (1,)), ((), ())),
        preferred_element_type=jnp.float32,
    )
    out_ref[...] = (o + b2_ref[0])[:, None, :]


def _main(gr, age, xm, comb_tab, w1, w1m, b1, w2, b2, n_rows):
    grid = (n_rows // _BLK,)
    return pl.pallas_call(
        _main_kernel,
        grid=grid,
        in_specs=[
            pl.BlockSpec((_BLK,), lambda i: (i,)),
            pl.BlockSpec((_BLK,), lambda i: (i,)),
            pl.BlockSpec((_BLK, 128), lambda i: (i, 0)),
            pl.BlockSpec((128, 256), lambda i: (0, 0)),
            pl.BlockSpec((256, 256), lambda i: (0, 0)),
            pl.BlockSpec((256, 128), lambda i: (0, 0)),
            pl.BlockSpec((1, 256), lambda i: (0, 0)),
            pl.BlockSpec((1, 256), lambda i: (0, 0)),
            pl.BlockSpec(memory_space=pltpu.SMEM),
        ],
        out_specs=pl.BlockSpec((1, 1, _BLK), lambda i: (i, 0, 0)),
        out_shape=jax.ShapeDtypeStruct((n_rows // _BLK, 1, _BLK), jnp.float32),
    )(gr, age, xm, comb_tab, w1, w1m, b1, w2, b2)


def kernel(gender, age, major, grade, gender_tab, age_tab, major_tab,
           grade_tab, W1, b1, W2, b2):
    n_rows = gender.shape[0]
    gender = gender.astype(jnp.int32)
    age = age.astype(jnp.int32)
    major = major.astype(jnp.int32)
    grade = grade.astype(jnp.int32)

    comb_tab = jnp.zeros((128, 256), jnp.float32)
    comb_tab = comb_tab.at[0:16, 0:64].set(jnp.repeat(gender_tab, 8, axis=0))
    comb_tab = comb_tab.at[0:16, 192:256].set(jnp.tile(grade_tab, (2, 1)))
    comb_tab = comb_tab.at[16:116, 64:128].set(age_tab)
    gr = gender * 8 + grade

    major_tab_p = jnp.pad(major_tab, ((0, 0), (0, 64)))
    xm = _sc_gather(major_tab_p, major, n_rows, 128)
    w1m = jnp.pad(W1[:, 128:192], ((0, 0), (0, 64)))
    out = _main(gr, age, xm, comb_tab, W1, w1m,
                b1.reshape(1, 256), W2, b2, n_rows)
    return out.reshape(n_rows)


# speedup vs baseline: 1.0380x; 1.0380x over previous
import functools

import jax
import jax.numpy as jnp
from jax import lax
from jax.experimental import pallas as pl
from jax.experimental.pallas import tpu as pltpu
from jax.experimental.pallas import tpu_sc as plsc

_NC = 2
_NS = 16
_BLK = 4096


def _sc_gather(table, idx, n_rows, dim):
    nw = _NC * _NS
    b_per_w = n_rows // nw
    mesh = plsc.VectorSubcoreMesh(core_axis_name="c", subcore_axis_name="s")

    @functools.partial(
        pl.kernel,
        mesh=mesh,
        out_type=jax.ShapeDtypeStruct((n_rows, dim), jnp.float32),
        scratch_types=[
            pltpu.VMEM((b_per_w,), jnp.int32),
            pltpu.VMEM((b_per_w, dim), jnp.float32),
            pltpu.SemaphoreType.DMA,
        ],
    )
    def gather_kernel(table_hbm, idx_hbm, out_hbm, idx_v, rows_v, sem):
        wid = lax.axis_index("s") * _NC + lax.axis_index("c")
        base = wid * b_per_w
        pltpu.sync_copy(idx_hbm.at[pl.ds(base, b_per_w)], idx_v)
        pltpu.async_copy(table_hbm.at[idx_v], rows_v, sem).wait()
        pltpu.sync_copy(rows_v, out_hbm.at[pl.ds(base, b_per_w)])

    return gather_kernel(table, idx)


def _main_kernel(gr_ref, a_ref, xm_ref, ct_ref, w1_ref, w1m_ref,
                 b1_ref, w2_ref, b2_ref, out_ref):
    pcomb = lax.dot_general(
        ct_ref[...], w1_ref[...], (((1,), (1,)), ((), ())),
        preferred_element_type=jnp.float32,
    )
    row = lax.broadcasted_iota(jnp.int32, pcomb.shape, 0)
    pcomb = pcomb + jnp.where(row < 16, b1_ref[...], 0.0)

    gr = gr_ref[...]
    a = a_ref[...]
    lane = lax.broadcasted_iota(jnp.int32, (_BLK, 128), 1)
    onehot = (lane == gr[:, None]) | (lane == a[:, None] + 16)
    m = onehot.astype(jnp.float32)
    h = lax.dot_general(
        m, pcomb, (((1,), (0,)), ((), ())),
        preferred_element_type=jnp.float32,
    )
    h = h + lax.dot_general(
        xm_ref[...], w1m_ref[...], (((1,), (1,)), ((), ())),
        preferred_element_type=jnp.float32,
    )
    h = jnp.maximum(h, 0.0)
    o = lax.dot_general(
        w2_ref[...], h, (((1,), (1,)), ((), ())),
        preferred_element_type=jnp.float32,
    )
    out_ref[...] = (o + b2_ref[0])[:, None, :]


def _main(gr, age, xm, comb_tab, w1, w1m, b1, w2, b2, n_rows):
    grid = (n_rows // _BLK,)
    return pl.pallas_call(
        _main_kernel,
        grid=grid,
        in_specs=[
            pl.BlockSpec((_BLK,), lambda i: (i,)),
            pl.BlockSpec((_BLK,), lambda i: (i,)),
            pl.BlockSpec((_BLK, 128), lambda i: (i, 0)),
            pl.BlockSpec((128, 256), lambda i: (0, 0)),
            pl.BlockSpec((256, 256), lambda i: (0, 0)),
            pl.BlockSpec((256, 128), lambda i: (0, 0)),
            pl.BlockSpec((1, 256), lambda i: (0, 0)),
            pl.BlockSpec((1, 256), lambda i: (0, 0)),
            pl.BlockSpec(memory_space=pltpu.SMEM),
        ],
        out_specs=pl.BlockSpec((1, 1, _BLK), lambda i: (i, 0, 0)),
        out_shape=jax.ShapeDtypeStruct((n_rows // _BLK, 1, _BLK), jnp.float32),
    )(gr, age, xm, comb_tab, w1, w1m, b1, w2, b2)


def kernel(gender, age, major, grade, gender_tab, age_tab, major_tab,
           grade_tab, W1, b1, W2, b2):
    n_rows = gender.shape[0]
    gender = gender.astype(jnp.int32)
    age = age.astype(jnp.int32)
    major = major.astype(jnp.int32)
    grade = grade.astype(jnp.int32)

    comb_tab = jnp.zeros((128, 256), jnp.float32)
    comb_tab = comb_tab.at[0:16, 0:64].set(jnp.repeat(gender_tab, 8, axis=0))
    comb_tab = comb_tab.at[0:16, 192:256].set(jnp.tile(grade_tab, (2, 1)))
    comb_tab = comb_tab.at[16:116, 64:128].set(age_tab)
    gr = gender * 8 + grade

    major_tab_p = jnp.pad(major_tab, ((0, 0), (0, 64)))
    xm = _sc_gather(major_tab_p, major, n_rows, 128)
    w1m = jnp.pad(W1[:, 128:192], ((0, 0), (0, 64)))
    out = _main(gr, age, xm, comb_tab, W1, w1m,
                b1.reshape(1, 256), W2, b2, n_rows)
    return out.reshape(n_rows)


# speedup vs baseline: 1.0420x; 1.0038x over previous
import functools

import jax
import jax.numpy as jnp
from jax import lax
from jax.experimental import pallas as pl
from jax.experimental.pallas import tpu as pltpu
from jax.experimental.pallas import tpu_sc as plsc

_NC = 2
_NS = 16
_BLK = 8192


def _sc_gather(table, idx, n_rows, dim):
    nw = _NC * _NS
    b_per_w = n_rows // nw
    mesh = plsc.VectorSubcoreMesh(core_axis_name="c", subcore_axis_name="s")

    @functools.partial(
        pl.kernel,
        mesh=mesh,
        out_type=jax.ShapeDtypeStruct((n_rows, dim), jnp.float32),
        scratch_types=[
            pltpu.VMEM((b_per_w,), jnp.int32),
            pltpu.VMEM((b_per_w, dim), jnp.float32),
            pltpu.SemaphoreType.DMA,
        ],
    )
    def gather_kernel(table_hbm, idx_hbm, out_hbm, idx_v, rows_v, sem):
        wid = lax.axis_index("s") * _NC + lax.axis_index("c")
        base = wid * b_per_w
        pltpu.sync_copy(idx_hbm.at[pl.ds(base, b_per_w)], idx_v)
        pltpu.async_copy(table_hbm.at[idx_v], rows_v, sem).wait()
        pltpu.sync_copy(rows_v, out_hbm.at[pl.ds(base, b_per_w)])

    return gather_kernel(table, idx)


def _main_kernel(gr_ref, a_ref, xm_ref, ct_ref, w1_ref, w1m_ref,
                 b1_ref, w2_ref, b2_ref, out_ref):
    pcomb = lax.dot_general(
        ct_ref[...], w1_ref[...], (((1,), (1,)), ((), ())),
        preferred_element_type=jnp.float32,
    )
    row = lax.broadcasted_iota(jnp.int32, pcomb.shape, 0)
    pcomb = pcomb + jnp.where(row < 16, b1_ref[...], 0.0)

    gr = gr_ref[...]
    a = a_ref[...]
    lane = lax.broadcasted_iota(jnp.int32, (_BLK, 128), 1)
    onehot = (lane == gr[:, None]) | (lane == a[:, None] + 16)
    m = onehot.astype(jnp.float32)
    h = lax.dot_general(
        m, pcomb, (((1,), (0,)), ((), ())),
        preferred_element_type=jnp.float32,
    )
    h = h + lax.dot_general(
        xm_ref[...], w1m_ref[...], (((1,), (1,)), ((), ())),
        preferred_element_type=jnp.float32,
    )
    h = jnp.maximum(h, 0.0)
    o = lax.dot_general(
        w2_ref[...], h, (((1,), (1,)), ((), ())),
        preferred_element_type=jnp.float32,
    )
    out_ref[...] = (o + b2_ref[0])[:, None, :]


def _main(gr, age, xm, comb_tab, w1, w1m, b1, w2, b2, n_rows):
    grid = (n_rows // _BLK,)
    return pl.pallas_call(
        _main_kernel,
        grid=grid,
        in_specs=[
            pl.BlockSpec((_BLK,), lambda i: (i,)),
            pl.BlockSpec((_BLK,), lambda i: (i,)),
            pl.BlockSpec((_BLK, 128), lambda i: (i, 0)),
            pl.BlockSpec((128, 256), lambda i: (0, 0)),
            pl.BlockSpec((256, 256), lambda i: (0, 0)),
            pl.BlockSpec((256, 128), lambda i: (0, 0)),
            pl.BlockSpec((1, 256), lambda i: (0, 0)),
            pl.BlockSpec((1, 256), lambda i: (0, 0)),
            pl.BlockSpec(memory_space=pltpu.SMEM),
        ],
        out_specs=pl.BlockSpec((1, 1, _BLK), lambda i: (i, 0, 0)),
        out_shape=jax.ShapeDtypeStruct((n_rows // _BLK, 1, _BLK), jnp.float32),
    )(gr, age, xm, comb_tab, w1, w1m, b1, w2, b2)


def kernel(gender, age, major, grade, gender_tab, age_tab, major_tab,
           grade_tab, W1, b1, W2, b2):
    n_rows = gender.shape[0]
    gender = gender.astype(jnp.int32)
    age = age.astype(jnp.int32)
    major = major.astype(jnp.int32)
    grade = grade.astype(jnp.int32)

    comb_tab = jnp.zeros((128, 256), jnp.float32)
    comb_tab = comb_tab.at[0:16, 0:64].set(jnp.repeat(gender_tab, 8, axis=0))
    comb_tab = comb_tab.at[0:16, 192:256].set(jnp.tile(grade_tab, (2, 1)))
    comb_tab = comb_tab.at[16:116, 64:128].set(age_tab)
    gr = gender * 8 + grade

    major_tab_p = jnp.pad(major_tab, ((0, 0), (0, 64)))
    xm = _sc_gather(major_tab_p, major, n_rows, 128)
    w1m = jnp.pad(W1[:, 128:192], ((0, 0), (0, 64)))
    out = _main(gr, age, xm, comb_tab, W1, w1m,
                b1.reshape(1, 256), W2, b2, n_rows)
    return out.reshape(n_rows)
